# baseline (speedup 1.0000x reference)
import functools

import jax
import jax.numpy as jnp
from jax import lax
from jax.experimental import pallas as pl
from jax.experimental.pallas import tpu as pltpu
from jax.experimental.pallas import tpu_sc as plsc

NC = 2
NS = 16
NW = NC * NS

B = 4096 * 200
D = 32
BPW = B // NW

G = 128
K = 8
CHUNK = K * G
NCHUNKS = BPW // CHUNK
IDX_ROWS_PER_W = BPW // G

_mesh = plsc.VectorSubcoreMesh(core_axis_name="c", subcore_axis_name="s")


@functools.partial(
    pl.kernel,
    mesh=_mesh,
    compiler_params=pltpu.CompilerParams(use_tc_tiling_on_sc=False),
    out_type=jax.ShapeDtypeStruct((B, D), jnp.float32),
    scratch_types=[
        pltpu.VMEM((IDX_ROWS_PER_W, G), jnp.int32),
        pltpu.VMEM((3, CHUNK, D), jnp.float32),
        pltpu.SemaphoreType.DMA,
        pltpu.SemaphoreType.DMA,
        pltpu.SemaphoreType.DMA,
        pltpu.SemaphoreType.DMA,
        pltpu.SemaphoreType.DMA,
        pltpu.SemaphoreType.DMA,
    ],
)
def _sc_gather(
    idx_hbm, table_hbm, out_hbm, idx_v, rows_v, g0, g1, g2, w0, w1, w2
):
    wid = lax.axis_index("s") * NC + lax.axis_index("c")
    idx_row0 = wid * IDX_ROWS_PER_W
    out_row0 = wid * BPW
    gsems = (g0, g1, g2)
    wsems = (w0, w1, w2)

    def start_gather(j, s):
        for r in range(K):
            pltpu.async_copy(
                table_hbm.at[idx_v.at[j * K + r]],
                rows_v.at[s, pl.ds(r * G, G)],
                gsems[s],
            )

    def wait_gather(s):
        pltpu.make_async_copy(
            out_hbm.at[pl.ds(0, CHUNK)], rows_v.at[s], gsems[s]
        ).wait()

    def start_write(j, s):
        pltpu.async_copy(
            rows_v.at[s], out_hbm.at[pl.ds(out_row0 + j * CHUNK, CHUNK)], wsems[s]
        )

    def wait_write(s):
        pltpu.make_async_copy(
            rows_v.at[s], out_hbm.at[pl.ds(0, CHUNK)], wsems[s]
        ).wait()

    pltpu.sync_copy(
        idx_hbm.at[pl.ds(idx_row0, IDX_ROWS_PER_W)], idx_v
    )

    start_gather(0, 0)

    start_gather(1, 1)
    wait_gather(0)
    start_write(0, 0)

    start_gather(2, 2)
    wait_gather(1)
    start_write(1, 1)

    @pl.loop(0, 7)
    def _(t):
        j = 2 + 3 * t
        wait_write(0)
        start_gather(j + 1, 0)
        wait_gather(2)
        start_write(j, 2)
        wait_write(1)
        start_gather(j + 2, 1)
        wait_gather(0)
        start_write(j + 1, 0)
        wait_write(2)
        start_gather(j + 3, 2)
        wait_gather(1)
        start_write(j + 2, 1)

    wait_write(0)
    start_gather(24, 0)
    wait_gather(2)
    start_write(23, 2)

    wait_gather(0)
    start_write(24, 0)

    wait_write(1)
    wait_write(2)
    wait_write(0)


def kernel(words, word_table):
    flat_idx = words.reshape(B // G, G)
    out = _sc_gather(flat_idx, word_table)
    return out.reshape(*words.shape, D)
